# baseline (speedup 1.0000x reference)
import functools

import jax
import jax.numpy as jnp
from jax import lax
from jax.experimental import pallas as pl
from jax.experimental.pallas import tpu as pltpu
from jax.experimental.pallas import tpu_sc as plsc

NUM_EMBEDDINGS = 100000
EMBEDDING_DIM = 128

_INFO = plsc.get_sparse_core_info()
_NW = _INFO.num_cores * _INFO.num_subcores

_CHUNK = 128
_NBUF = 5
_LOOK = 3


def _embed_kernel(b_per_w, n_chunks, table_hbm, ids_hbm, out_hbm,
                  idx_v, rows_v, gsem, wsem):
    wid = lax.axis_index("c") * _INFO.num_subcores + lax.axis_index("s")
    base = wid * b_per_w
    pltpu.sync_copy(ids_hbm.at[pl.ds(base, b_per_w)], idx_v)

    def g_copy(i, b):
        return pltpu.make_async_copy(
            table_hbm.at[idx_v.at[pl.ds(i * _CHUNK, _CHUNK)]],
            rows_v.at[b], gsem.at[b])

    def w_copy(i, b):
        return pltpu.make_async_copy(
            rows_v.at[b], out_hbm.at[pl.ds(base + i * _CHUNK, _CHUNK)],
            wsem.at[b])

    for b in range(_LOOK):
        g_copy(b, b).start()

    n_groups = n_chunks // _NBUF

    def group(g, _):
        i0 = g * _NBUF
        for b in range(_NBUF):
            i = i0 + b
            g_copy(i, b).wait()
            w_copy(i, b).start()
            k = i + _LOOK
            kb = (b + _LOOK) % _NBUF

            def issue_next(kk):
                w_copy(kk - _NBUF, kb).wait()
                g_copy(kk, kb).start()

            def issue_first(kk):
                g_copy(kk, kb).start()

            if b + _LOOK < _NBUF:
                pl.when(g > 0)(lambda: issue_next(k))
                pl.when(g == 0)(lambda: issue_first(k))
            else:
                pl.when(g < n_groups - 1)(lambda: issue_next(k))
        return 0

    lax.fori_loop(0, n_groups, group, 0)

    for b in range(_NBUF):
        w_copy(n_chunks - _NBUF + b, b).wait()


def kernel(input_ids, table):
    B = input_ids.shape[0] * input_ids.shape[1]
    assert B % (_NW * _CHUNK * _NBUF) == 0
    b_per_w = B // _NW
    n_chunks = b_per_w // _CHUNK
    ids_flat = input_ids.reshape(B).astype(jnp.int32)

    mesh = plsc.VectorSubcoreMesh(core_axis_name="c", subcore_axis_name="s")
    run = pl.kernel(
        functools.partial(_embed_kernel, b_per_w, n_chunks),
        mesh=mesh,
        out_type=jax.ShapeDtypeStruct((B, EMBEDDING_DIM), jnp.float32),
        scratch_types=[
            pltpu.VMEM((b_per_w,), jnp.int32),
            pltpu.VMEM((_NBUF, _CHUNK, EMBEDDING_DIM), jnp.float32),
            pltpu.SemaphoreType.DMA((_NBUF,)),
            pltpu.SemaphoreType.DMA((_NBUF,)),
        ],
    )
    out = run(table, ids_flat)
    return out.reshape(input_ids.shape[0], input_ids.shape[1], EMBEDDING_DIM)


# speedup vs baseline: 1.0053x; 1.0053x over previous
import functools

import jax
import jax.numpy as jnp
from jax import lax
from jax.experimental import pallas as pl
from jax.experimental.pallas import tpu as pltpu
from jax.experimental.pallas import tpu_sc as plsc

NUM_EMBEDDINGS = 100000
EMBEDDING_DIM = 128

_INFO = plsc.get_sparse_core_info()
_NW = _INFO.num_cores * _INFO.num_subcores

_CHUNK = 128
_NBUF = 5
_LOOK = 3


def _embed_kernel(b_per_w, n_chunks, table_hbm, ids_hbm, out_hbm,
                  idx_v, rows_v, gsem, wsem):
    wid = lax.axis_index("s") * _INFO.num_cores + lax.axis_index("c")
    base = wid * b_per_w
    pltpu.sync_copy(ids_hbm.at[pl.ds(base, b_per_w)], idx_v)

    def g_copy(i, b):
        return pltpu.make_async_copy(
            table_hbm.at[idx_v.at[pl.ds(i * _CHUNK, _CHUNK)]],
            rows_v.at[b], gsem.at[b])

    def w_copy(i, b):
        return pltpu.make_async_copy(
            rows_v.at[b], out_hbm.at[pl.ds(base + i * _CHUNK, _CHUNK)],
            wsem.at[b])

    for b in range(_LOOK):
        g_copy(b, b).start()

    n_groups = n_chunks // _NBUF

    def group(g, _):
        i0 = g * _NBUF
        for b in range(_NBUF):
            i = i0 + b
            g_copy(i, b).wait()
            w_copy(i, b).start()
            k = i + _LOOK
            kb = (b + _LOOK) % _NBUF

            def issue_next(kk):
                w_copy(kk - _NBUF, kb).wait()
                g_copy(kk, kb).start()

            def issue_first(kk):
                g_copy(kk, kb).start()

            if b + _LOOK < _NBUF:
                pl.when(g > 0)(lambda: issue_next(k))
                pl.when(g == 0)(lambda: issue_first(k))
            else:
                pl.when(g < n_groups - 1)(lambda: issue_next(k))
        return 0

    lax.fori_loop(0, n_groups, group, 0)

    for b in range(_NBUF):
        w_copy(n_chunks - _NBUF + b, b).wait()


def kernel(input_ids, table):
    B = input_ids.shape[0] * input_ids.shape[1]
    assert B % (_NW * _CHUNK * _NBUF) == 0
    b_per_w = B // _NW
    n_chunks = b_per_w // _CHUNK
    ids_flat = input_ids.reshape(B).astype(jnp.int32)

    mesh = plsc.VectorSubcoreMesh(core_axis_name="c", subcore_axis_name="s")
    run = pl.kernel(
        functools.partial(_embed_kernel, b_per_w, n_chunks),
        mesh=mesh,
        out_type=jax.ShapeDtypeStruct((B, EMBEDDING_DIM), jnp.float32),
        scratch_types=[
            pltpu.VMEM((b_per_w,), jnp.int32),
            pltpu.VMEM((_NBUF, _CHUNK, EMBEDDING_DIM), jnp.float32),
            pltpu.SemaphoreType.DMA((_NBUF,)),
            pltpu.SemaphoreType.DMA((_NBUF,)),
        ],
    )
    out = run(table, ids_flat)
    return out.reshape(input_ids.shape[0], input_ids.shape[1], EMBEDDING_DIM)
